# baseline (speedup 1.0000x reference)
import functools

import jax
import jax.numpy as jnp
from jax import lax
from jax.experimental import pallas as pl
from jax.experimental.pallas import tpu as pltpu
from jax.experimental.pallas import tpu_sc as plsc

_B, _P, _S, _H, _N = 64, 10, 20, 128, 128
_BP = _B * _P
_L = 16
_NW = 32
_CHUNKS = _BP // _L
_ROWS_PER_W = _B // 8
_AGENT_VREGS = _B // _L


def _sc_body(starts_hbm, ends_hbm, valid_hbm, npxt_hbm, npyt_hbm, p2b_hbm,
             nctx_hbm, lx_hbm, ly_hbm,
             nctx_out, mask_out, npxg_out, npyg_out,
             starts_v, ends_v, valid_v, npxt_v, npyt_v, p2b_v, rows_v,
             lx_v, ly_v, outm_v, npg_v, sem):
    wid = lax.axis_index("s") * 2 + lax.axis_index("c")

    pltpu.sync_copy(starts_hbm, starts_v)
    pltpu.sync_copy(ends_hbm, ends_v)
    pltpu.sync_copy(valid_hbm, valid_v)
    pltpu.sync_copy(npxt_hbm, npxt_v)
    pltpu.sync_copy(npyt_hbm, npyt_v)
    pltpu.sync_copy(p2b_hbm, p2b_v)

    @pl.when(wid < 8)
    def _gather_rows():
        row_base = wid * _ROWS_PER_W
        idx = starts_v.at[pl.ds(row_base, _ROWS_PER_W)]
        pltpu.async_copy(nctx_hbm.at[idx], rows_v, sem).wait()
        pltpu.sync_copy(rows_v, nctx_out.at[pl.ds(row_base, _ROWS_PER_W)])

    @pl.when(jnp.logical_and(wid >= 8, wid < 8 + _AGENT_VREGS))
    def _gather_npos():
        j = wid - 8
        ids = lax.broadcasted_iota(jnp.int32, (_L,), 0) + j * _L
        sidx = plsc.load_gather(starts_v, [ids])
        npg_v[0] = plsc.load_gather(npxt_v, [sidx])
        pltpu.sync_copy(npg_v, npxg_out.at[j])
        npg_v[0] = plsc.load_gather(npyt_v, [sidx])
        pltpu.sync_copy(npg_v, npyg_out.at[j])

    def _do_chunk(c):
        pltpu.sync_copy(lx_hbm.at[c], lx_v)
        pltpu.sync_copy(ly_hbm.at[c], ly_v)
        pair = lax.broadcasted_iota(jnp.int32, (_L,), 0) + c * _L
        b_ids = plsc.load_gather(p2b_v, [pair])
        sidx = plsc.load_gather(starts_v, [b_ids])
        eidx = plsc.load_gather(ends_v, [b_ids])
        vld = plsc.load_gather(valid_v, [b_ids])
        npx = plsc.load_gather(npxt_v, [sidx])
        npy = plsc.load_gather(npyt_v, [sidx])
        cond = jnp.logical_and(vld > 0, (eidx - sidx) > 0)
        d2min = jnp.full((_L,), 3.4e38, jnp.float32)
        nan_any = jnp.zeros((_L,), jnp.bool_)
        for s in range(_S):
            lxs = lx_v[s]
            lys = ly_v[s]
            nan_any = nan_any | (lxs != lxs) | (lys != lys)
            dx = npx - lxs
            dy = npy - lys
            d2min = jnp.minimum(d2min, dx * dx + dy * dy)
        d2 = jnp.where(nan_any, npx * npx + npy * npy, d2min)
        keep = jnp.logical_and(cond, d2 < 10000.0)
        outm_v[0] = jnp.where(keep, 1.0, 0.0).astype(jnp.float32)
        pltpu.sync_copy(outm_v, mask_out.at[c])

    _do_chunk(wid)

    @pl.when(wid < _CHUNKS - _NW)
    def _second_round():
        _do_chunk(wid + _NW)


_sc_kernel = functools.partial(
    pl.kernel,
    out_type=[
        jax.ShapeDtypeStruct((_B, _H), jnp.float32),
        jax.ShapeDtypeStruct((_CHUNKS, 1, _L), jnp.float32),
        jax.ShapeDtypeStruct((_AGENT_VREGS, 1, _L), jnp.float32),
        jax.ShapeDtypeStruct((_AGENT_VREGS, 1, _L), jnp.float32),
    ],
    mesh=plsc.VectorSubcoreMesh(core_axis_name="c", subcore_axis_name="s"),
    compiler_params=pltpu.CompilerParams(use_tc_tiling_on_sc=False,
                                         needs_layout_passes=False),
    scratch_types=[
        pltpu.VMEM((_B,), jnp.int32),
        pltpu.VMEM((_B,), jnp.int32),
        pltpu.VMEM((_B,), jnp.int32),
        pltpu.VMEM((_N,), jnp.float32),
        pltpu.VMEM((_N,), jnp.float32),
        pltpu.VMEM((_BP,), jnp.int32),
        pltpu.VMEM((_ROWS_PER_W, _H), jnp.float32),
        pltpu.VMEM((_S, _L), jnp.float32),
        pltpu.VMEM((_S, _L), jnp.float32),
        pltpu.VMEM((1, _L), jnp.float32),
        pltpu.VMEM((1, _L), jnp.float32),
        pltpu.SemaphoreType.DMA,
    ],
)(_sc_body)


def _tc_body(nctx_ref, mask_ref, npx_ref, npy_ref, actx_ref,
             Wn_ref, Wa_ref, Wp_ref, Wih_ref, Whh_ref,
             bm_ref, bi_ref, bh_ref, out_ref):
    H = _H
    nctx = nctx_ref[...]
    actx = actx_ref[...]
    npx = npx_ref[...]
    npy = npy_ref[...]
    xg = (jnp.dot(nctx, Wn_ref[...], preferred_element_type=jnp.float32)
          + jnp.dot(actx, Wa_ref[...], preferred_element_type=jnp.float32)
          + (-npx) * Wp_ref[0:1, :] + (-npy) * Wp_ref[1:2, :]
          + bm_ref[...])
    x = jnp.maximum(xg, 0.0)
    gi = jnp.dot(x, Wih_ref[...], preferred_element_type=jnp.float32) + bi_ref[...]
    gh = jnp.dot(nctx, Whh_ref[...], preferred_element_type=jnp.float32) + bh_ref[...]
    r_g = jax.nn.sigmoid(gi[:, :H] + gh[:, :H])
    z = jax.nn.sigmoid(gi[:, H:2 * H] + gh[:, H:2 * H])
    n_g = jnp.tanh(gi[:, 2 * H:] + r_g * gh[:, 2 * H:])
    r = (1.0 - z) * n_g + z * nctx
    for l in range(_P):
        out_ref[:, l, :] = jnp.where(mask_ref[:, l:l + 1] > 0.5, r, 0.0)


def kernel(agent_pos, agent_context, ngh_pos, ngh_context, possible_lanes,
           lane_context, label, seq_start_end, valid_neighbor,
           W_msg, b_msg, W_ih, W_hh, b_ih, b_hh):
    B, P, H = lane_context.shape
    S = possible_lanes.shape[0]

    starts = seq_start_end[:, 0]
    ends = seq_start_end[:, 1]
    valid_i = valid_neighbor.astype(jnp.int32)
    npx_tab = ngh_pos[:, 0]
    npy_tab = ngh_pos[:, 1]
    pair2b = (jnp.arange(_BP, dtype=jnp.int32) // P)
    lanes_x = possible_lanes[:, :, 0].reshape(S, _CHUNKS, _L).transpose(1, 0, 2)
    lanes_y = possible_lanes[:, :, 1].reshape(S, _CHUNKS, _L).transpose(1, 0, 2)

    nctx_g, mask3, npxg3, npyg3 = _sc_kernel(
        starts, ends, valid_i, npx_tab, npy_tab, pair2b, ngh_context,
        lanes_x, lanes_y)

    mask = mask3.reshape(B, P)
    Wp = W_msg[:, :2].T
    Wn = W_msg[:, 2:2 + H].T
    Wa = W_msg[:, 2 + H:].T
    out2 = pl.pallas_call(
        _tc_body,
        out_shape=jax.ShapeDtypeStruct((B, P, H), jnp.float32),
    )(nctx_g, mask, npxg3.reshape(B, 1), npyg3.reshape(B, 1), agent_context,
      Wn, Wa, Wp, W_ih.T, W_hh.T,
      b_msg.reshape(1, H), b_ih.reshape(1, 3 * H), b_hh.reshape(1, 3 * H))

    return (lane_context, out2)


# speedup vs baseline: 2.8364x; 2.8364x over previous
import functools

import jax
import jax.numpy as jnp
from jax.experimental import pallas as pl


def _dn(a, b):
    return jax.lax.dot_general(a, b, (((1,), (1,)), ((), ())),
                               preferred_element_type=jnp.float32)


def _body(B, P, S, H, N,
          sse_ref, valid_ref, actx_ref, nctx_tab_ref, npos_tab_ref,
          lx_ref, ly_ref, Wm_ref, Wih_ref, Whh_ref,
          bm_ref, bi_ref, bh_ref, out_ref):
    starts = sse_ref[:, 0:1]
    ends = sse_ref[:, 1:2]
    iota_n = jax.lax.broadcasted_iota(jnp.int32, (B, N), 1)
    onehot = (iota_n == starts).astype(jnp.float32)

    nctx = jnp.dot(onehot, nctx_tab_ref[...],
                   preferred_element_type=jnp.float32)
    nposg = jnp.dot(onehot, npos_tab_ref[...],
                    preferred_element_type=jnp.float32)
    npx = nposg[:, 0:1]
    npy = nposg[:, 1:2]

    actx = actx_ref[...]
    Wm = Wm_ref[...]
    xg = (_dn(nctx, Wm[:, 2:2 + H]) + _dn(actx, Wm[:, 2 + H:])
          + _dn(-nposg, Wm[:, 0:2]) + bm_ref[...])
    x = jnp.maximum(xg, 0.0)

    gi = _dn(x, Wih_ref[...]) + bi_ref[...]
    gh = _dn(nctx, Whh_ref[...]) + bh_ref[...]
    r_g = jax.nn.sigmoid(gi[:, :H] + gh[:, :H])
    z = jax.nn.sigmoid(gi[:, H:2 * H] + gh[:, H:2 * H])
    n_g = jnp.tanh(gi[:, 2 * H:] + r_g * gh[:, 2 * H:])
    r = (1.0 - z) * n_g + z * nctx

    cond = jnp.logical_and(valid_ref[...] > 0, (ends - starts) > 0)

    d2min = jnp.full((B, P), jnp.inf, jnp.float32)
    nan_any = jnp.zeros((B, P), jnp.bool_)
    for s in range(S):
        lxs = lx_ref[s]
        lys = ly_ref[s]
        nan_any = nan_any | jnp.isnan(lxs) | jnp.isnan(lys)
        dx = npx - lxs
        dy = npy - lys
        d2min = jnp.minimum(d2min, dx * dx + dy * dy)
    d2 = jnp.where(nan_any, npx * npx + npy * npy, d2min)
    keep = cond & (d2 < 10000.0)

    for l in range(P):
        out_ref[:, l, :] = jnp.where(keep[:, l:l + 1], r, 0.0)


def kernel(agent_pos, agent_context, ngh_pos, ngh_context, possible_lanes,
           lane_context, label, seq_start_end, valid_neighbor,
           W_msg, b_msg, W_ih, W_hh, b_ih, b_hh):
    B, P, H = lane_context.shape
    S = possible_lanes.shape[0]
    N = ngh_context.shape[0]

    lx = possible_lanes[:, :, 0].reshape(S, B, P)
    ly = possible_lanes[:, :, 1].reshape(S, B, P)
    valid_i = valid_neighbor.astype(jnp.int32).reshape(B, 1)

    body = functools.partial(_body, B, P, S, H, N)
    out2 = pl.pallas_call(
        body,
        out_shape=jax.ShapeDtypeStruct((B, P, H), jnp.float32),
    )(seq_start_end, valid_i, agent_context, ngh_context, ngh_pos,
      lx, ly, W_msg, W_ih, W_hh, b_msg, b_ih, b_hh)

    return (lane_context, out2)
